# baseline (speedup 1.0000x reference)
import functools

import jax
import jax.numpy as jnp
from jax import lax
from jax.experimental import pallas as pl
from jax.experimental.pallas import tpu as pltpu, tpu_sc as plsc
import numpy as np

N = 4
LQ = 4096
H = 64
W = 64
LIN = H * W
D_MODEL = 256
M = 8
P = 4
DH = D_MODEL // M

NQ = N * LQ
RT = NQ * M
TAPS = 4 * P

_HIGH = jax.lax.Precision.HIGHEST


_MM_BLK = 1024


def _mm_kernel(x_ref, w_ref, b_ref, o_ref):
    o_ref[...] = (
        jnp.dot(x_ref[...].astype(jnp.bfloat16), w_ref[...].astype(jnp.bfloat16),
                preferred_element_type=jnp.float32)
        + b_ref[...]
    )


def _matmul_bias(x, w, b):
    rows = x.shape[0]
    return pl.pallas_call(
        _mm_kernel,
        grid=(rows // _MM_BLK,),
        in_specs=[
            pl.BlockSpec((_MM_BLK, x.shape[1]), lambda i: (i, 0)),
            pl.BlockSpec(w.shape, lambda i: (0, 0)),
            pl.BlockSpec((1, b.shape[0]), lambda i: (0, 0)),
        ],
        out_specs=pl.BlockSpec((_MM_BLK, w.shape[1]), lambda i: (i, 0)),
        out_shape=jax.ShapeDtypeStruct((rows, w.shape[1]), jnp.float32),
    )(x, w, b.reshape(1, -1))


def _mm_pack_kernel(x_ref, w_ref, b_ref, o_ref):
    y = (jnp.dot(x_ref[...].astype(jnp.bfloat16), w_ref[...].astype(jnp.bfloat16),
                 preferred_element_type=jnp.float32)
         + b_ref[...])
    u = lax.bitcast_convert_type(y, jnp.uint32)
    c16 = jnp.uint32(16)
    bf = lax.shift_right_logical(
        u + jnp.uint32(0x7FFF) + (lax.shift_right_logical(u, c16) & jnp.uint32(1)),
        c16)
    d2 = y.shape[1] // 2
    word = lax.shift_left(bf[:, d2:], c16) | bf[:, :d2]
    o_ref[...] = lax.bitcast_convert_type(word, jnp.int32)


def _matmul_pack(x, w, b):
    rows = x.shape[0]
    return pl.pallas_call(
        _mm_pack_kernel,
        grid=(rows // _MM_BLK,),
        in_specs=[
            pl.BlockSpec((_MM_BLK, x.shape[1]), lambda i: (i, 0)),
            pl.BlockSpec(w.shape, lambda i: (0, 0)),
            pl.BlockSpec((1, b.shape[0]), lambda i: (0, 0)),
        ],
        out_specs=pl.BlockSpec((_MM_BLK, w.shape[1] // 2), lambda i: (i, 0)),
        out_shape=jax.ShapeDtypeStruct((rows, w.shape[1] // 2), jnp.int32),
    )(x, w, b.reshape(1, -1))


_PREP_BLK = 512


def _prep_kernel(q_ref, rx_ref, ry_ref, ws_ref, bs_ref, sx_ref, sy_ref,
                 wa_ref, ba_ref, g_ref, idx_ref, w_ref):
    q = q_ref[...]
    off = jnp.dot(q, ws_ref[...], preferred_element_type=jnp.float32) + bs_ref[...]
    offx = jnp.dot(off, sx_ref[...], precision=_HIGH,
                   preferred_element_type=jnp.float32)
    offy = jnp.dot(off, sy_ref[...], precision=_HIGH,
                   preferred_element_type=jnp.float32)
    a = jnp.dot(q, wa_ref[...], preferred_element_type=jnp.float32) + ba_ref[...]
    ea = jnp.exp(a - jnp.max(a, axis=-1, keepdims=True))
    s = jnp.dot(ea, g_ref[...], precision=_HIGH, preferred_element_type=jnp.float32)
    attw = ea / s

    ix = rx_ref[...] + offx
    iy = ry_ref[...] + offy
    ix0 = jnp.floor(ix)
    iy0 = jnp.floor(iy)
    fx = ix - ix0
    fy = iy - iy0

    cat4 = lambda x: jnp.concatenate([x, x, x, x], axis=1)
    ix0w = cat4(ix0)
    iy0w = cat4(iy0)
    fxw = cat4(fx)
    fyw = cat4(fy)
    aww = cat4(attw)
    lane = jax.lax.broadcasted_iota(jnp.int32, ix0w.shape, 1)
    dx = ((lane // 32) & 1).astype(jnp.float32)
    dy = (lane // 64).astype(jnp.float32)
    mf = ((lane % 32) // 4).astype(jnp.float32)
    n_off = lax.convert_element_type(
        (pl.program_id(0) // (LQ // _PREP_BLK)) * (LIN * M), jnp.float32)

    xk = ix0w + dx
    yk = iy0w + dy
    wx = (1.0 - fxw) + dx * (fxw + fxw - 1.0)
    wy = (1.0 - fyw) + dy * (fyw + fyw - 1.0)
    valid = ((xk >= 0.0) & (xk <= W - 1.0)
             & (yk >= 0.0) & (yk <= H - 1.0)).astype(jnp.float32)
    pos = jnp.clip(yk, 0.0, H - 1.0) * float(W) + jnp.clip(xk, 0.0, W - 1.0)
    idx_ref[...] = ((pos * float(M) + mf) + n_off).astype(jnp.int32)
    w_ref[...] = aww * wx * wy * valid


def _group_sum_matrix():
    g = np.zeros((32, 32), np.float32)
    for i in range(32):
        for j in range(32):
            if i // 4 == j // 4:
                g[i, j] = 1.0
    return jnp.asarray(g)


def _xy_select_matrices():
    sx = np.zeros((M * P * 2, M * P), np.float32)
    sy = np.zeros((M * P * 2, M * P), np.float32)
    for m in range(M):
        for p in range(P):
            sx[m * (P * 2) + p * 2, m * P + p] = 1.0
            sy[m * (P * 2) + p * 2 + 1, m * P + p] = 1.0
    return jnp.asarray(sx), jnp.asarray(sy)


def _prep(qf, rx, ry, ws, bs2, wa, ba):
    nblk = NQ // _PREP_BLK
    full = lambda arr: pl.BlockSpec(arr.shape, lambda i: (0, 0))
    g = _group_sum_matrix()
    sx, sy = _xy_select_matrices()
    return pl.pallas_call(
        _prep_kernel,
        grid=(nblk,),
        in_specs=[
            pl.BlockSpec((_PREP_BLK, D_MODEL), lambda i: (i, 0)),
            pl.BlockSpec((_PREP_BLK, 1), lambda i: (i, 0)),
            pl.BlockSpec((_PREP_BLK, 1), lambda i: (i, 0)),
            full(ws), full(bs2), full(sx), full(sy),
            full(wa), full(ba), full(g),
        ],
        out_specs=[
            pl.BlockSpec((_PREP_BLK, M * TAPS), lambda i: (i, 0)),
            pl.BlockSpec((_PREP_BLK, M * TAPS), lambda i: (i, 0)),
        ],
        out_shape=[
            jax.ShapeDtypeStruct((NQ, M * TAPS), jnp.int32),
            jax.ShapeDtypeStruct((NQ, M * TAPS), jnp.float32),
        ],
    )(qf, rx, ry, ws, bs2, sx, sy, wa, ba, g)


_NW = 32
_QW = NQ // _NW
_CQ = 16
_CR = _CQ * M
_CT = _CR * TAPS
_NCH = _QW // _CQ

_SPLAT_DNUMS = jax.lax.GatherDimensionNumbers(
    offset_dims=(), collapsed_slice_dims=(0,), start_index_map=(0,))


def _splat(vec16, k):
    idx = jnp.full((16,), k, jnp.int32)
    return jax.lax.gather(vec16, idx[:, None], _SPLAT_DNUMS, (1,),
                          mode=jax.lax.GatherScatterMode.PROMISE_IN_BOUNDS)


def _sc_body(tab, idxh, wh, out,
             idx_v0, idx_v1, idx_v2, idx_v3, w_v0, w_v1, w_v2, w_v3,
             rows_v0, rows_v1, out_v,
             sem0, sem1, iw_sem0, iw_sem1, iw_sem2, iw_sem3):
    cid = lax.axis_index("c")
    sid = lax.axis_index("s")
    wid = sid * 2 + cid
    q_base = wid * _QW
    idx_vs = (idx_v0, idx_v1, idx_v2, idx_v3)
    w_vs = (w_v0, w_v1, w_v2, w_v3)
    rows_vs = (rows_v0, rows_v1)
    sems = (sem0, sem1)
    iw_sems = (iw_sem0, iw_sem1, iw_sem2, iw_sem3)

    def stage_iw(ci, iwb):
        q0 = q_base + ci * _CQ
        pltpu.async_copy(idxh.at[pl.ds(q0, _CQ)], idx_vs[iwb], iw_sems[iwb])
        pltpu.async_copy(wh.at[pl.ds(q0, _CQ)], w_vs[iwb], iw_sems[iwb])

    def fire(ci, iwb, rb):
        q0 = q_base + ci * _CQ
        pltpu.make_async_copy(idxh.at[pl.ds(q0, _CQ)], idx_vs[iwb],
                              iw_sems[iwb]).wait()
        pltpu.make_async_copy(wh.at[pl.ds(q0, _CQ)], w_vs[iwb],
                              iw_sems[iwb]).wait()
        for b in range(_CT // 128):
            pltpu.async_copy(tab.at[idx_vs[iwb].at[b]],
                             rows_vs[rb].at[pl.ds(b * 128, 128)], sems[rb])

    def drain(iwb, rb):
        for b in range(_CT // 128):
            pltpu.make_async_copy(tab.at[idx_vs[iwb].at[b]],
                                  rows_vs[rb].at[pl.ds(b * 128, 128)],
                                  sems[rb]).wait()

    def compute(ci, iwb, rb):
        w_v = w_vs[iwb]
        rows_v = rows_vs[rb]

        def rowj(j, carry2):
            base = j * (M * TAPS)
            wv = [[w_v[j, pl.ds(t * 32 + h * 16, 16)] for h in range(2)]
                  for t in range(4)]
            for m in range(M):
                h, lm = m // 4, (m % 4) * 4
                acc0 = jnp.zeros((16,), jnp.float32)
                acc1 = jnp.zeros((16,), jnp.float32)
                for t in range(4):
                    for p in range(4):
                        wk = _splat(wv[t][h], lm + p)
                        r = rows_v[base + t * 32 + m * 4 + p, :]
                        lo = lax.bitcast_convert_type(lax.shift_left(r, 16),
                                                      jnp.float32)
                        hi = lax.bitcast_convert_type(r & jnp.int32(-65536),
                                                      jnp.float32)
                        acc0 = acc0 + wk * lo
                        acc1 = acc1 + wk * hi
                out_v[j, pl.ds(m * DH, 16)] = acc0
                out_v[j, pl.ds(m * DH + 16, 16)] = acc1
            return carry2

        lax.fori_loop(0, _CQ, rowj, 0)
        pltpu.sync_copy(out_v, out.at[pl.ds(q_base + ci * _CQ, _CQ)])

    for k in range(4):
        stage_iw(k, k)
    fire(0, 0, 0)
    fire(1, 1, 1)

    def quad(i, carry):
        ci0 = i * 4
        for k in range(4):
            c = ci0 + k
            rb = k % 2
            drain(k, rb)
            compute(c, k, rb)

            @pl.when(c + 4 < _NCH)
            def _():
                stage_iw(c + 4, k)

            @pl.when(c + 2 < _NCH)
            def _():
                fire(c + 2, (k + 2) % 4, rb)

        return carry

    lax.fori_loop(0, _NCH // 4, quad, 0)


def _sc_gather(tab, idx, w):
    mesh = plsc.VectorSubcoreMesh(core_axis_name="c", subcore_axis_name="s")
    fn = functools.partial(
        pl.kernel,
        out_type=jax.ShapeDtypeStruct((NQ, D_MODEL), jnp.float32),
        mesh=mesh,
        compiler_params=pltpu.CompilerParams(use_tc_tiling_on_sc=False),
        scratch_types=(
            [pltpu.VMEM((_CQ, M * TAPS), jnp.int32)] * 4
            + [pltpu.VMEM((_CQ, M * TAPS), jnp.float32)] * 4
            + [pltpu.VMEM((_CT, DH // 2), jnp.int32)] * 2
            + [pltpu.VMEM((_CQ, D_MODEL), jnp.float32)]
            + [pltpu.SemaphoreType.DMA] * 6
        ),
    )(_sc_body)
    return fn(tab, idx, w)


def kernel(query, reference_points, input_flatten, input_spatial_shapes,
           Wv, bv, Ws, bs, Wa, ba, Wo, bo):
    qf = query.reshape(NQ, D_MODEL)
    xf = input_flatten.reshape(N * LIN, D_MODEL)

    cperm = (np.arange(M)[:, None] * DH + np.arange(DH // 2)[None, :]).reshape(-1)
    cperm = np.concatenate([cperm, cperm + DH // 2])
    value = _matmul_pack(xf, Wv[:, cperm], bv[cperm])
    tab = value.reshape(RT, DH // 2)

    refxy = reference_points.reshape(NQ, 2)
    rx = refxy[:, 0:1] * float(W) - 0.5
    ry = refxy[:, 1:2] * float(H) - 0.5

    idx, wgt = _prep(qf, rx, ry, Ws, bs.reshape(1, -1), Wa, ba.reshape(1, -1))

    sampled = _sc_gather(tab, idx, wgt)

    out = _matmul_bias(sampled, Wo, bo)
    return out.reshape(N, LQ, D_MODEL)


# speedup vs baseline: 1.0534x; 1.0534x over previous
import functools

import jax
import jax.numpy as jnp
from jax import lax
from jax.experimental import pallas as pl
from jax.experimental.pallas import tpu as pltpu, tpu_sc as plsc
import numpy as np

N = 4
LQ = 4096
H = 64
W = 64
LIN = H * W
D_MODEL = 256
M = 8
P = 4
DH = D_MODEL // M

NQ = N * LQ
RT = NQ * M
TAPS = 4 * P

_HIGH = jax.lax.Precision.HIGHEST


_MM_BLK = 1024


def _mm_kernel(x_ref, w_ref, b_ref, o_ref):
    o_ref[...] = (
        jnp.dot(x_ref[...].astype(jnp.bfloat16), w_ref[...].astype(jnp.bfloat16),
                preferred_element_type=jnp.float32)
        + b_ref[...]
    )


def _matmul_bias(x, w, b):
    rows = x.shape[0]
    return pl.pallas_call(
        _mm_kernel,
        grid=(rows // _MM_BLK,),
        in_specs=[
            pl.BlockSpec((_MM_BLK, x.shape[1]), lambda i: (i, 0)),
            pl.BlockSpec(w.shape, lambda i: (0, 0)),
            pl.BlockSpec((1, b.shape[0]), lambda i: (0, 0)),
        ],
        out_specs=pl.BlockSpec((_MM_BLK, w.shape[1]), lambda i: (i, 0)),
        out_shape=jax.ShapeDtypeStruct((rows, w.shape[1]), jnp.float32),
    )(x, w, b.reshape(1, -1))


def _mm_pack_kernel(x_ref, w_ref, b_ref, o_ref):
    y = (jnp.dot(x_ref[...].astype(jnp.bfloat16), w_ref[...].astype(jnp.bfloat16),
                 preferred_element_type=jnp.float32)
         + b_ref[...])
    u = lax.bitcast_convert_type(y, jnp.uint32)
    c16 = jnp.uint32(16)
    bf = lax.shift_right_logical(
        u + jnp.uint32(0x7FFF) + (lax.shift_right_logical(u, c16) & jnp.uint32(1)),
        c16)
    d2 = y.shape[1] // 2
    word = lax.shift_left(bf[:, d2:], c16) | bf[:, :d2]
    o_ref[...] = lax.bitcast_convert_type(word, jnp.int32)


def _matmul_pack(x, w, b):
    rows = x.shape[0]
    return pl.pallas_call(
        _mm_pack_kernel,
        grid=(rows // _MM_BLK,),
        in_specs=[
            pl.BlockSpec((_MM_BLK, x.shape[1]), lambda i: (i, 0)),
            pl.BlockSpec(w.shape, lambda i: (0, 0)),
            pl.BlockSpec((1, b.shape[0]), lambda i: (0, 0)),
        ],
        out_specs=pl.BlockSpec((_MM_BLK, w.shape[1] // 2), lambda i: (i, 0)),
        out_shape=jax.ShapeDtypeStruct((rows, w.shape[1] // 2), jnp.int32),
    )(x, w, b.reshape(1, -1))


_PREP_BLK = 512


def _prep_kernel(q_ref, rx_ref, ry_ref, wsx_ref, bsx_ref, wsy_ref, bsy_ref,
                 wa_ref, ba_ref, g_ref, idx_ref, w_ref):
    q = q_ref[...]
    offx = jnp.dot(q, wsx_ref[...], preferred_element_type=jnp.float32) + bsx_ref[...]
    offy = jnp.dot(q, wsy_ref[...], preferred_element_type=jnp.float32) + bsy_ref[...]
    a = jnp.dot(q, wa_ref[...], preferred_element_type=jnp.float32) + ba_ref[...]
    ea = jnp.exp(a - jnp.max(a, axis=-1, keepdims=True))
    s = jnp.dot(ea, g_ref[...], precision=_HIGH, preferred_element_type=jnp.float32)
    attw = ea / s

    ix = rx_ref[...] + offx
    iy = ry_ref[...] + offy
    ix0 = jnp.floor(ix)
    iy0 = jnp.floor(iy)
    fx = ix - ix0
    fy = iy - iy0

    lane = jax.lax.broadcasted_iota(jnp.int32, ix.shape, 1)
    mf = (lane // 4).astype(jnp.float32)
    n_off = lax.convert_element_type(
        (pl.program_id(0) // (LQ // _PREP_BLK)) * (LIN * M), jnp.float32)

    idx_taps = []
    w_taps = []
    for dy in (0.0, 1.0):
        for dx in (0.0, 1.0):
            xk = ix0 + dx
            yk = iy0 + dy
            valid = ((xk >= 0.0) & (xk <= W - 1.0)
                     & (yk >= 0.0) & (yk <= H - 1.0)).astype(jnp.float32)
            wt = (fx if dx else 1.0 - fx) * (fy if dy else 1.0 - fy)
            xc = jnp.clip(xk, 0.0, W - 1.0)
            yc = jnp.clip(yk, 0.0, H - 1.0)
            pos = yc * float(W) + xc
            idx_taps.append((pos * float(M) + mf) + n_off)
            w_taps.append(attw * wt * valid)

    idx_ref[...] = jnp.concatenate(idx_taps, axis=1).astype(jnp.int32)
    w_ref[...] = jnp.concatenate(w_taps, axis=1)


def _group_sum_matrix():
    g = np.zeros((32, 32), np.float32)
    for i in range(32):
        for j in range(32):
            if i // 4 == j // 4:
                g[i, j] = 1.0
    return jnp.asarray(g)


def _prep(qf, rx, ry, wsx, bsx, wsy, bsy, wa, ba):
    nblk = NQ // _PREP_BLK
    full = lambda arr: pl.BlockSpec(arr.shape, lambda i: (0, 0))
    g = _group_sum_matrix()
    return pl.pallas_call(
        _prep_kernel,
        grid=(nblk,),
        in_specs=[
            pl.BlockSpec((_PREP_BLK, D_MODEL), lambda i: (i, 0)),
            pl.BlockSpec((_PREP_BLK, 1), lambda i: (i, 0)),
            pl.BlockSpec((_PREP_BLK, 1), lambda i: (i, 0)),
            full(wsx), full(bsx), full(wsy), full(bsy),
            full(wa), full(ba), full(g),
        ],
        out_specs=[
            pl.BlockSpec((_PREP_BLK, M * TAPS), lambda i: (i, 0)),
            pl.BlockSpec((_PREP_BLK, M * TAPS), lambda i: (i, 0)),
        ],
        out_shape=[
            jax.ShapeDtypeStruct((NQ, M * TAPS), jnp.int32),
            jax.ShapeDtypeStruct((NQ, M * TAPS), jnp.float32),
        ],
    )(qf, rx, ry, wsx, bsx, wsy, bsy, wa, ba, g)


_NW = 32
_QW = NQ // _NW
_CQ = 16
_CR = _CQ * M
_CT = _CR * TAPS
_NCH = _QW // _CQ

_SPLAT_DNUMS = jax.lax.GatherDimensionNumbers(
    offset_dims=(), collapsed_slice_dims=(0,), start_index_map=(0,))


def _splat(vec16, k):
    idx = jnp.full((16,), k, jnp.int32)
    return jax.lax.gather(vec16, idx[:, None], _SPLAT_DNUMS, (1,),
                          mode=jax.lax.GatherScatterMode.PROMISE_IN_BOUNDS)


def _sc_body(tab, idxh, wh, out,
             idx_v0, idx_v1, idx_v2, idx_v3, w_v0, w_v1, w_v2, w_v3,
             rows_v0, rows_v1, out_v,
             sem0, sem1, iw_sem0, iw_sem1, iw_sem2, iw_sem3):
    cid = lax.axis_index("c")
    sid = lax.axis_index("s")
    wid = sid * 2 + cid
    q_base = wid * _QW
    idx_vs = (idx_v0, idx_v1, idx_v2, idx_v3)
    w_vs = (w_v0, w_v1, w_v2, w_v3)
    rows_vs = (rows_v0, rows_v1)
    sems = (sem0, sem1)
    iw_sems = (iw_sem0, iw_sem1, iw_sem2, iw_sem3)

    def stage_iw(ci, iwb):
        q0 = q_base + ci * _CQ
        pltpu.async_copy(idxh.at[pl.ds(q0, _CQ)], idx_vs[iwb], iw_sems[iwb])
        pltpu.async_copy(wh.at[pl.ds(q0, _CQ)], w_vs[iwb], iw_sems[iwb])

    def fire(ci, iwb, rb):
        q0 = q_base + ci * _CQ
        pltpu.make_async_copy(idxh.at[pl.ds(q0, _CQ)], idx_vs[iwb],
                              iw_sems[iwb]).wait()
        pltpu.make_async_copy(wh.at[pl.ds(q0, _CQ)], w_vs[iwb],
                              iw_sems[iwb]).wait()
        for b in range(_CT // 128):
            pltpu.async_copy(tab.at[idx_vs[iwb].at[b]],
                             rows_vs[rb].at[pl.ds(b * 128, 128)], sems[rb])

    def drain(iwb, rb):
        for b in range(_CT // 128):
            pltpu.make_async_copy(tab.at[idx_vs[iwb].at[b]],
                                  rows_vs[rb].at[pl.ds(b * 128, 128)],
                                  sems[rb]).wait()

    def compute(ci, iwb, rb):
        w_v = w_vs[iwb]
        rows_v = rows_vs[rb]

        def rowj(j, carry2):
            base = j * (M * TAPS)
            wv = [[w_v[j, pl.ds(t * 32 + h * 16, 16)] for h in range(2)]
                  for t in range(4)]
            for m in range(M):
                h, lm = m // 4, (m % 4) * 4
                acc0 = jnp.zeros((16,), jnp.float32)
                acc1 = jnp.zeros((16,), jnp.float32)
                for t in range(4):
                    for p in range(4):
                        wk = _splat(wv[t][h], lm + p)
                        r = rows_v[base + t * 32 + m * 4 + p, :]
                        lo = lax.bitcast_convert_type(lax.shift_left(r, 16),
                                                      jnp.float32)
                        hi = lax.bitcast_convert_type(r & jnp.int32(-65536),
                                                      jnp.float32)
                        acc0 = acc0 + wk * lo
                        acc1 = acc1 + wk * hi
                out_v[j, pl.ds(m * DH, 16)] = acc0
                out_v[j, pl.ds(m * DH + 16, 16)] = acc1
            return carry2

        lax.fori_loop(0, _CQ, rowj, 0)
        pltpu.sync_copy(out_v, out.at[pl.ds(q_base + ci * _CQ, _CQ)])

    for k in range(4):
        stage_iw(k, k)
    fire(0, 0, 0)
    fire(1, 1, 1)

    def quad(i, carry):
        ci0 = i * 4
        for k in range(4):
            c = ci0 + k
            rb = k % 2
            drain(k, rb)
            compute(c, k, rb)

            @pl.when(c + 4 < _NCH)
            def _():
                stage_iw(c + 4, k)

            @pl.when(c + 2 < _NCH)
            def _():
                fire(c + 2, (k + 2) % 4, rb)

        return carry

    lax.fori_loop(0, _NCH // 4, quad, 0)


def _sc_gather(tab, idx, w):
    mesh = plsc.VectorSubcoreMesh(core_axis_name="c", subcore_axis_name="s")
    fn = functools.partial(
        pl.kernel,
        out_type=jax.ShapeDtypeStruct((NQ, D_MODEL), jnp.float32),
        mesh=mesh,
        compiler_params=pltpu.CompilerParams(use_tc_tiling_on_sc=False),
        scratch_types=(
            [pltpu.VMEM((_CQ, M * TAPS), jnp.int32)] * 4
            + [pltpu.VMEM((_CQ, M * TAPS), jnp.float32)] * 4
            + [pltpu.VMEM((_CT, DH // 2), jnp.int32)] * 2
            + [pltpu.VMEM((_CQ, D_MODEL), jnp.float32)]
            + [pltpu.SemaphoreType.DMA] * 6
        ),
    )(_sc_body)
    return fn(tab, idx, w)


def kernel(query, reference_points, input_flatten, input_spatial_shapes,
           Wv, bv, Ws, bs, Wa, ba, Wo, bo):
    qf = query.reshape(NQ, D_MODEL)
    xf = input_flatten.reshape(N * LIN, D_MODEL)

    cperm = (np.arange(M)[:, None] * DH + np.arange(DH // 2)[None, :]).reshape(-1)
    cperm = np.concatenate([cperm, cperm + DH // 2])
    value = _matmul_pack(xf, Wv[:, cperm], bv[cperm])
    tab = value.reshape(RT, DH // 2)

    refxy = reference_points.reshape(NQ, 2)
    rx = refxy[:, 0:1] * float(W) - 0.5
    ry = refxy[:, 1:2] * float(H) - 0.5
    wsx = Ws[:, 0::2]
    wsy = Ws[:, 1::2]
    bsx = bs[0::2].reshape(1, -1)
    bsy = bs[1::2].reshape(1, -1)
    ba2 = ba.reshape(1, -1)

    idx, wgt = _prep(qf, rx, ry, wsx, bsx, wsy, bsy, Wa, ba2)

    sampled = _sc_gather(tab, idx, wgt)

    out = _matmul_bias(sampled, Wo, bo)
    return out.reshape(N, LQ, D_MODEL)
